# baseline (speedup 1.0000x reference)
import functools

import jax
import jax.numpy as jnp
from jax import lax
from jax.experimental import pallas as pl
from jax.experimental.pallas import tpu as pltpu
from jax.experimental.pallas import tpu_sc as plsc

N = 4096
B = 256
NNZ = 167772

BLKC = 128
NBLK = N // BLKC
BLK_ELEMS = N * BLKC
SLICE = BLK_ELEMS // 16
SPAD = 1024
ZB = 16384

PER16 = ((NNZ + 16 * 128 - 1) // (16 * 128)) * 128
NNZ_PAD = 16 * PER16
NROWS = PER16 // 128

SPLITS = (11, 5)

_mesh = plsc.VectorSubcoreMesh(core_axis_name="c", subcore_axis_name="s")


def _make_densify(start_blk, bpc):
    def _densify(vals_hbm, rows_hbm, cols_hbm, w_hbm,
                 rows_v, cols_v, vals_v, idx_full, bbuf, zbuf, sblk_a, sblk_b,
                 sem, semz, semc):
        c = lax.axis_index("c")
        s = lax.axis_index("s")
        base = s * PER16
        a1 = pltpu.async_copy(rows_hbm.at[pl.ds(base, PER16)], rows_v, sem)
        a2 = pltpu.async_copy(cols_hbm.at[pl.ds(base, PER16)], cols_v, sem)
        a3 = pltpu.async_copy(vals_hbm.at[pl.ds(base, PER16)], vals_v, sem)
        a1.wait()
        a2.wait()
        a3.wait()

        zero16 = jnp.zeros((16,), jnp.float32)

        def _zb(i, carry):
            zbuf[pl.ds(i * 16, 16)] = zero16
            return carry

        lax.fori_loop(0, ZB // 16, _zb, 0)

        dump = BLK_ELEMS + s * 64

        def _first_row(pred):
            def step(_, lohi):
                lo, hi = lohi
                mid = (lo + hi) >> 1
                p = pred(jnp.minimum(mid, NROWS - 1)) | (lo >= hi)
                return (jnp.where(p, lo, mid + 1), jnp.where(p, mid, hi))

            lo, _ = lax.fori_loop(0, 7, step,
                                  (jnp.int32(0), jnp.int32(NROWS)))
            return lo

        first_blk = start_blk + c * bpc
        a0 = _first_row(
            lambda r: cols_v[pl.ds(r * 128, 16)][0] >= first_blk * BLKC)
        rz = _first_row(
            lambda r: cols_v[pl.ds(r * 128, 16)][0] >= (first_blk + bpc) * BLKC)

        def _pre(r, carry):
            for kk in range(8):
                off = r * 128 + kk * 16
                rr = rows_v[pl.ds(off, 16)]
                cc = cols_v[pl.ds(off, 16)]
                idx_full[r, pl.ds(kk * 16, 16)] = rr * BLKC + (cc & (BLKC - 1))
            return carry

        lax.fori_loop(jnp.maximum(a0 - 1, 0), rz, _pre, 0)

        def _zero_fire(buf, i, sem_):
            pltpu.async_copy(zbuf, buf.at[pl.ds(s * SLICE + i * ZB, ZB)], sem_)

        def _zero_drain(buf, i, sem_):
            pltpu.make_async_copy(
                zbuf, buf.at[pl.ds(s * SLICE + i * ZB, ZB)], sem_).wait()

        def _scatter(buf, blk, a):
            b2 = _first_row(
                lambda r: cols_v[pl.ds(r * 128, 16)][0] >= (blk + 1) * BLKC)
            interior_end = jnp.maximum(b2 - 1, a)

            def _if(r, carry):
                pltpu.async_copy(vals_v.at[pl.ds(r * 128, 128)],
                                 buf.at[idx_full.at[r]], sem, add=True)
                return carry

            def _id(r, carry):
                pltpu.make_async_copy(vals_v.at[pl.ds(r * 128, 128)],
                                      buf.at[idx_full.at[r]], sem).wait()
                return carry

            lax.fori_loop(a, interior_end, _if, 0)

            def _brow(r):
                for kk in range(8):
                    off = r * 128 + kk * 16
                    cc = cols_v[pl.ds(off, 16)]
                    valid = (cc >> 7) == blk
                    iv = idx_full[r, pl.ds(kk * 16, 16)]
                    bbuf[pl.ds(kk * 16, 16)] = jnp.where(valid, iv, dump)
                pltpu.sync_copy(vals_v.at[pl.ds(r * 128, 128)],
                                buf.at[bbuf], add=True)

            @pl.when(a > 0)
            def _():
                _brow(a - 1)

            @pl.when(b2 > a)
            def _():
                _brow(b2 - 1)

            lax.fori_loop(a, interior_end, _id, 0)
            return b2

        def _out_slices(j):
            lb = c * bpc + j
            blk = start_blk + lb
            return blk, lb * BLK_ELEMS + s * SLICE

        for i in range(SLICE // ZB):
            _zero_fire(sblk_a, i, semz)
        for i in range(SLICE // ZB):
            _zero_drain(sblk_a, i, semz)
        plsc.subcore_barrier()

        a_chain = a0
        for j in range(bpc):
            cur, nxt = (sblk_a, sblk_b) if j % 2 == 0 else (sblk_b, sblk_a)
            blk, w_off = _out_slices(j)
            if j > 0:
                _, w_off_prev = _out_slices(j - 1)
                pltpu.async_copy(nxt.at[pl.ds(s * SLICE, SLICE)],
                                 w_hbm.at[pl.ds(w_off_prev, SLICE)], semc)
            a_chain = _scatter(cur, blk, a_chain)
            if j > 0:
                pltpu.make_async_copy(nxt.at[pl.ds(s * SLICE, SLICE)],
                                      w_hbm.at[pl.ds(w_off_prev, SLICE)],
                                      semc).wait()
            if j < bpc - 1:
                for i in range(SLICE // ZB):
                    _zero_fire(nxt, i, semz)
                for i in range(SLICE // ZB):
                    _zero_drain(nxt, i, semz)
            plsc.subcore_barrier()

        last = sblk_a if (bpc - 1) % 2 == 0 else sblk_b
        _, w_off_last = _out_slices(bpc - 1)
        pltpu.sync_copy(last.at[pl.ds(s * SLICE, SLICE)],
                        w_hbm.at[pl.ds(w_off_last, SLICE)])

    return functools.partial(
        pl.kernel,
        out_type=jax.ShapeDtypeStruct((2 * bpc * BLK_ELEMS,), jnp.float32),
        mesh=_mesh,
        scratch_types=[
            pltpu.VMEM((PER16,), jnp.int32),
            pltpu.VMEM((PER16,), jnp.int32),
            pltpu.VMEM((PER16,), jnp.float32),
            pltpu.VMEM((NROWS, 128), jnp.int32),
            pltpu.VMEM((128,), jnp.int32),
            pltpu.VMEM((ZB,), jnp.float32),
            pltpu.VMEM_SHARED((BLK_ELEMS + SPAD,), jnp.float32),
            pltpu.VMEM_SHARED((BLK_ELEMS + SPAD,), jnp.float32),
            pltpu.SemaphoreType.DMA,
            pltpu.SemaphoreType.DMA,
            pltpu.SemaphoreType.DMA,
        ],
    )(_densify)


_densify_calls = tuple(
    _make_densify(sum(2 * b for b in SPLITS[:i]), SPLITS[i])
    for i in range(len(SPLITS)))


def _mm_body(x_ref, w_ref, o_ref):
    w = w_ref[...].reshape(N, BLKC)
    o_ref[...] = jnp.maximum(
        jnp.dot(x_ref[...], w, preferred_element_type=jnp.float32), 0.0)


def _matmul_relu(x, w_flat):
    nb = w_flat.shape[0] // BLK_ELEMS
    return pl.pallas_call(
        _mm_body,
        grid=(nb,),
        in_specs=[
            pl.BlockSpec((B, N), lambda i: (0, 0)),
            pl.BlockSpec((BLK_ELEMS,), lambda i: (i,)),
        ],
        out_specs=pl.BlockSpec((B, BLKC), lambda i: (0, i)),
        out_shape=jax.ShapeDtypeStruct((B, nb * BLKC), jnp.float32),
    )(x, w_flat)


def kernel(inputs, values, row_idx, col_idx):
    pad = NNZ_PAD - NNZ
    vals_p = jnp.pad(values, (0, pad))
    rows_p = jnp.pad(row_idx, (0, pad))
    cols_p = jnp.pad(col_idx, (0, pad), constant_values=N)
    outs = []
    for fn in _densify_calls:
        w_flat = fn(vals_p, rows_p, cols_p)
        outs.append(_matmul_relu(inputs, w_flat))
    return jnp.concatenate(outs, axis=1)


# speedup vs baseline: 1.0459x; 1.0459x over previous
import functools

import jax
import jax.numpy as jnp
from jax import lax
from jax.experimental import pallas as pl
from jax.experimental.pallas import tpu as pltpu
from jax.experimental.pallas import tpu_sc as plsc

N = 4096
B = 256
NNZ = 167772

BLKC = 128
NBLK = N // BLKC
BLK_ELEMS = N * BLKC
SLICE = BLK_ELEMS // 16
SPAD = 1024
ZB = 16384

PER16 = ((NNZ + 16 * 128 - 1) // (16 * 128)) * 128
NNZ_PAD = 16 * PER16
NROWS = PER16 // 128

SPLITS = (8, 8)

_mesh = plsc.VectorSubcoreMesh(core_axis_name="c", subcore_axis_name="s")


def _make_densify(start_blk, bpc):
    def _densify(vals_hbm, rows_hbm, cols_hbm, w_hbm,
                 rows_v, cols_v, vals_v, idx_full, bbuf, zbuf, sblk_a, sblk_b,
                 sem, semz, semc):
        c = lax.axis_index("c")
        s = lax.axis_index("s")
        base = s * PER16
        a1 = pltpu.async_copy(rows_hbm.at[pl.ds(base, PER16)], rows_v, sem)
        a2 = pltpu.async_copy(cols_hbm.at[pl.ds(base, PER16)], cols_v, sem)
        a3 = pltpu.async_copy(vals_hbm.at[pl.ds(base, PER16)], vals_v, sem)
        a1.wait()
        a2.wait()
        a3.wait()

        zero16 = jnp.zeros((16,), jnp.float32)

        def _zb(i, carry):
            zbuf[pl.ds(i * 16, 16)] = zero16
            return carry

        lax.fori_loop(0, ZB // 16, _zb, 0)

        dump = BLK_ELEMS + s * 64

        def _first_row(pred):
            def step(_, lohi):
                lo, hi = lohi
                mid = (lo + hi) >> 1
                p = pred(jnp.minimum(mid, NROWS - 1)) | (lo >= hi)
                return (jnp.where(p, lo, mid + 1), jnp.where(p, mid, hi))

            lo, _ = lax.fori_loop(0, 7, step,
                                  (jnp.int32(0), jnp.int32(NROWS)))
            return lo

        first_blk = start_blk + c * bpc
        a0 = _first_row(
            lambda r: cols_v[pl.ds(r * 128, 16)][0] >= first_blk * BLKC)
        rz = _first_row(
            lambda r: cols_v[pl.ds(r * 128, 16)][0] >= (first_blk + bpc) * BLKC)

        def _pre(r, carry):
            for kk in range(8):
                off = r * 128 + kk * 16
                rr = rows_v[pl.ds(off, 16)]
                cc = cols_v[pl.ds(off, 16)]
                idx_full[r, pl.ds(kk * 16, 16)] = rr * BLKC + (cc & (BLKC - 1))
            return carry

        lax.fori_loop(jnp.maximum(a0 - 1, 0), rz, _pre, 0)

        def _zero_fire(buf, i, sem_):
            pltpu.async_copy(zbuf, buf.at[pl.ds(s * SLICE + i * ZB, ZB)], sem_)

        def _zero_drain(buf, i, sem_):
            pltpu.make_async_copy(
                zbuf, buf.at[pl.ds(s * SLICE + i * ZB, ZB)], sem_).wait()

        def _scatter(buf, blk, a):
            b2 = _first_row(
                lambda r: cols_v[pl.ds(r * 128, 16)][0] >= (blk + 1) * BLKC)
            interior_end = jnp.maximum(b2 - 1, a)

            def _if(r, carry):
                pltpu.async_copy(vals_v.at[pl.ds(r * 128, 128)],
                                 buf.at[idx_full.at[r]], sem, add=True)
                return carry

            def _id(r, carry):
                pltpu.make_async_copy(vals_v.at[pl.ds(r * 128, 128)],
                                      buf.at[idx_full.at[r]], sem).wait()
                return carry

            lax.fori_loop(a, interior_end, _if, 0)

            def _brow(r):
                for kk in range(8):
                    off = r * 128 + kk * 16
                    cc = cols_v[pl.ds(off, 16)]
                    valid = (cc >> 7) == blk
                    iv = idx_full[r, pl.ds(kk * 16, 16)]
                    bbuf[pl.ds(kk * 16, 16)] = jnp.where(valid, iv, dump)
                pltpu.sync_copy(vals_v.at[pl.ds(r * 128, 128)],
                                buf.at[bbuf], add=True)

            @pl.when(a > 0)
            def _():
                _brow(a - 1)

            @pl.when(b2 > a)
            def _():
                _brow(b2 - 1)

            lax.fori_loop(a, interior_end, _id, 0)
            return b2

        def _out_slices(j):
            lb = c * bpc + j
            blk = start_blk + lb
            return blk, lb * BLK_ELEMS + s * SLICE

        for i in range(SLICE // ZB):
            _zero_fire(sblk_a, i, semz)
        for i in range(SLICE // ZB):
            _zero_drain(sblk_a, i, semz)
        plsc.subcore_barrier()

        a_chain = a0
        for j in range(bpc):
            cur, nxt = (sblk_a, sblk_b) if j % 2 == 0 else (sblk_b, sblk_a)
            blk, w_off = _out_slices(j)
            if j > 0:
                _, w_off_prev = _out_slices(j - 1)
                pltpu.async_copy(nxt.at[pl.ds(s * SLICE, SLICE)],
                                 w_hbm.at[pl.ds(w_off_prev, SLICE)], semc)
            a_chain = _scatter(cur, blk, a_chain)
            if j > 0:
                pltpu.make_async_copy(nxt.at[pl.ds(s * SLICE, SLICE)],
                                      w_hbm.at[pl.ds(w_off_prev, SLICE)],
                                      semc).wait()
            if j < bpc - 1:
                for i in range(SLICE // ZB):
                    _zero_fire(nxt, i, semz)
                for i in range(SLICE // ZB):
                    _zero_drain(nxt, i, semz)
            plsc.subcore_barrier()

        last = sblk_a if (bpc - 1) % 2 == 0 else sblk_b
        _, w_off_last = _out_slices(bpc - 1)
        pltpu.sync_copy(last.at[pl.ds(s * SLICE, SLICE)],
                        w_hbm.at[pl.ds(w_off_last, SLICE)])

    return functools.partial(
        pl.kernel,
        out_type=jax.ShapeDtypeStruct((2 * bpc * BLK_ELEMS,), jnp.float32),
        mesh=_mesh,
        scratch_types=[
            pltpu.VMEM((PER16,), jnp.int32),
            pltpu.VMEM((PER16,), jnp.int32),
            pltpu.VMEM((PER16,), jnp.float32),
            pltpu.VMEM((NROWS, 128), jnp.int32),
            pltpu.VMEM((128,), jnp.int32),
            pltpu.VMEM((ZB,), jnp.float32),
            pltpu.VMEM_SHARED((BLK_ELEMS + SPAD,), jnp.float32),
            pltpu.VMEM_SHARED((BLK_ELEMS + SPAD,), jnp.float32),
            pltpu.SemaphoreType.DMA,
            pltpu.SemaphoreType.DMA,
            pltpu.SemaphoreType.DMA,
        ],
    )(_densify)


_densify_calls = tuple(
    _make_densify(sum(2 * b for b in SPLITS[:i]), SPLITS[i])
    for i in range(len(SPLITS)))


def _mm_body(x_ref, w_ref, o_ref):
    w = w_ref[...].reshape(N, BLKC)
    o_ref[...] = jnp.maximum(
        jnp.dot(x_ref[...], w, preferred_element_type=jnp.float32), 0.0)


def _matmul_relu(x, w_flat):
    nb = w_flat.shape[0] // BLK_ELEMS
    return pl.pallas_call(
        _mm_body,
        grid=(nb,),
        in_specs=[
            pl.BlockSpec((B, N), lambda i: (0, 0)),
            pl.BlockSpec((BLK_ELEMS,), lambda i: (i,)),
        ],
        out_specs=pl.BlockSpec((B, BLKC), lambda i: (0, i)),
        out_shape=jax.ShapeDtypeStruct((B, nb * BLKC), jnp.float32),
    )(x, w_flat)


def kernel(inputs, values, row_idx, col_idx):
    pad = NNZ_PAD - NNZ
    vals_p = jnp.pad(values, (0, pad))
    rows_p = jnp.pad(row_idx, (0, pad))
    cols_p = jnp.pad(col_idx, (0, pad), constant_values=N)
    outs = []
    for fn in _densify_calls:
        w_flat = fn(vals_p, rows_p, cols_p)
        outs.append(_matmul_relu(inputs, w_flat))
    return jnp.concatenate(outs, axis=1)


# speedup vs baseline: 1.0858x; 1.0382x over previous
import functools

import jax
import jax.numpy as jnp
from jax import lax
from jax.experimental import pallas as pl
from jax.experimental.pallas import tpu as pltpu
from jax.experimental.pallas import tpu_sc as plsc

N = 4096
B = 256
NNZ = 167772

BLKC = 128
NBLK = N // BLKC
BLK_ELEMS = N * BLKC
SLICE = BLK_ELEMS // 16
SPAD = 1024
ZB = 16384

PER16 = ((NNZ + 16 * 128 - 1) // (16 * 128)) * 128
NNZ_PAD = 16 * PER16
NROWS = PER16 // 128

SPLITS = (8, 8)

_mesh = plsc.VectorSubcoreMesh(core_axis_name="c", subcore_axis_name="s")


def _make_densify(start_blk, bpc):
    def _densify(vals_hbm, rows_hbm, cols_hbm, w_hbm,
                 rows_v, cols_v, vals_v, idx_full, bbuf, zbuf, sblk_a, sblk_b,
                 sem, semz, semc):
        c = lax.axis_index("c")
        s = lax.axis_index("s")
        base = s * PER16
        a1 = pltpu.async_copy(rows_hbm.at[pl.ds(base, PER16)], rows_v, sem)
        a2 = pltpu.async_copy(cols_hbm.at[pl.ds(base, PER16)], cols_v, sem)
        a3 = pltpu.async_copy(vals_hbm.at[pl.ds(base, PER16)], vals_v, sem)
        a1.wait()
        a2.wait()
        a3.wait()

        zero16 = jnp.zeros((16,), jnp.float32)

        def _zb(i, carry):
            zbuf[pl.ds(i * 16, 16)] = zero16
            return carry

        lax.fori_loop(0, ZB // 16, _zb, 0)

        dump = BLK_ELEMS + s * 64

        def _first_row(pred):
            def step(_, lohi):
                lo, hi = lohi
                mid = (lo + hi) >> 1
                p = pred(jnp.minimum(mid, NROWS - 1)) | (lo >= hi)
                return (jnp.where(p, lo, mid + 1), jnp.where(p, mid, hi))

            lo, _ = lax.fori_loop(0, 7, step,
                                  (jnp.int32(0), jnp.int32(NROWS)))
            return lo

        def _pre(r, carry):
            for kk in range(8):
                off = r * 128 + kk * 16
                rr = rows_v[pl.ds(off, 16)]
                cc = cols_v[pl.ds(off, 16)]
                idx_full[r, pl.ds(kk * 16, 16)] = rr * BLKC + (cc & (BLKC - 1))
            return carry

        lax.fori_loop(0, NROWS, _pre, 0)

        def _zero_fire(buf, i, sem_):
            pltpu.async_copy(zbuf, buf.at[pl.ds(s * SLICE + i * ZB, ZB)], sem_)

        def _zero_drain(buf, i, sem_):
            pltpu.make_async_copy(
                zbuf, buf.at[pl.ds(s * SLICE + i * ZB, ZB)], sem_).wait()

        def _scatter(buf, blk):
            a = _first_row(
                lambda r: cols_v[pl.ds(r * 128, 16)][0] >= blk * BLKC)
            b2 = _first_row(
                lambda r: cols_v[pl.ds(r * 128, 16)][0] >= (blk + 1) * BLKC)
            interior_end = jnp.maximum(b2 - 1, a)

            def _if(r, carry):
                pltpu.async_copy(vals_v.at[pl.ds(r * 128, 128)],
                                 buf.at[idx_full.at[r]], sem, add=True)
                return carry

            def _id(r, carry):
                pltpu.make_async_copy(vals_v.at[pl.ds(r * 128, 128)],
                                      buf.at[idx_full.at[r]], sem).wait()
                return carry

            lax.fori_loop(a, interior_end, _if, 0)

            def _brow(r):
                for kk in range(8):
                    off = r * 128 + kk * 16
                    cc = cols_v[pl.ds(off, 16)]
                    valid = (cc >> 7) == blk
                    iv = idx_full[r, pl.ds(kk * 16, 16)]
                    bbuf[pl.ds(kk * 16, 16)] = jnp.where(valid, iv, dump)
                pltpu.sync_copy(vals_v.at[pl.ds(r * 128, 128)],
                                buf.at[bbuf], add=True)

            @pl.when(a > 0)
            def _():
                _brow(a - 1)

            @pl.when(b2 > a)
            def _():
                _brow(b2 - 1)

            lax.fori_loop(a, interior_end, _id, 0)

        def _out_slices(j):
            lb = c * bpc + j
            blk = start_blk + lb
            return blk, lb * BLK_ELEMS + s * SLICE

        for i in range(SLICE // ZB):
            _zero_fire(sblk_a, i, semz)
        for i in range(SLICE // ZB):
            _zero_drain(sblk_a, i, semz)
        plsc.subcore_barrier()

        for j in range(bpc):
            cur, nxt = (sblk_a, sblk_b) if j % 2 == 0 else (sblk_b, sblk_a)
            blk, w_off = _out_slices(j)
            if j > 0:
                _, w_off_prev = _out_slices(j - 1)
                pltpu.async_copy(nxt.at[pl.ds(s * SLICE, SLICE)],
                                 w_hbm.at[pl.ds(w_off_prev, SLICE)], semc)
            _scatter(cur, blk)
            if j > 0:
                pltpu.make_async_copy(nxt.at[pl.ds(s * SLICE, SLICE)],
                                      w_hbm.at[pl.ds(w_off_prev, SLICE)],
                                      semc).wait()
            if j < bpc - 1:
                for i in range(SLICE // ZB):
                    _zero_fire(nxt, i, semz)
                for i in range(SLICE // ZB):
                    _zero_drain(nxt, i, semz)
            plsc.subcore_barrier()

        last = sblk_a if (bpc - 1) % 2 == 0 else sblk_b
        _, w_off_last = _out_slices(bpc - 1)
        pltpu.sync_copy(last.at[pl.ds(s * SLICE, SLICE)],
                        w_hbm.at[pl.ds(w_off_last, SLICE)])

    return functools.partial(
        pl.kernel,
        out_type=jax.ShapeDtypeStruct((2 * bpc * BLK_ELEMS,), jnp.float32),
        mesh=_mesh,
        scratch_types=[
            pltpu.VMEM((PER16,), jnp.int32),
            pltpu.VMEM((PER16,), jnp.int32),
            pltpu.VMEM((PER16,), jnp.float32),
            pltpu.VMEM((NROWS, 128), jnp.int32),
            pltpu.VMEM((128,), jnp.int32),
            pltpu.VMEM((ZB,), jnp.float32),
            pltpu.VMEM_SHARED((BLK_ELEMS + SPAD,), jnp.float32),
            pltpu.VMEM_SHARED((BLK_ELEMS + SPAD,), jnp.float32),
            pltpu.SemaphoreType.DMA,
            pltpu.SemaphoreType.DMA,
            pltpu.SemaphoreType.DMA,
        ],
    )(_densify)


_densify_calls = tuple(
    _make_densify(sum(2 * b for b in SPLITS[:i]), SPLITS[i])
    for i in range(len(SPLITS)))


def _mm_body(x_ref, w_ref, o_ref):
    w = w_ref[...].reshape(N, BLKC)
    o_ref[...] = jnp.maximum(
        jnp.dot(x_ref[...], w, preferred_element_type=jnp.float32), 0.0)


def _matmul_relu(x, w_flat):
    nb = w_flat.shape[0] // BLK_ELEMS
    return pl.pallas_call(
        _mm_body,
        grid=(nb,),
        in_specs=[
            pl.BlockSpec((B, N), lambda i: (0, 0)),
            pl.BlockSpec((BLK_ELEMS,), lambda i: (i,)),
        ],
        out_specs=pl.BlockSpec((B, BLKC), lambda i: (0, i)),
        out_shape=jax.ShapeDtypeStruct((B, nb * BLKC), jnp.float32),
    )(x, w_flat)


def kernel(inputs, values, row_idx, col_idx):
    pad = NNZ_PAD - NNZ
    vals_p = jnp.pad(values, (0, pad))
    rows_p = jnp.pad(row_idx, (0, pad))
    cols_p = jnp.pad(col_idx, (0, pad), constant_values=N)
    outs = []
    for fn in _densify_calls:
        w_flat = fn(vals_p, rows_p, cols_p)
        outs.append(_matmul_relu(inputs, w_flat))
    return jnp.concatenate(outs, axis=1)
